# Initial kernel scaffold; baseline (speedup 1.0000x reference)
import functools

import jax
import jax.numpy as jnp
from jax import lax
from jax.experimental import pallas as pl
from jax.experimental.pallas import tpu as pltpu
from jax.experimental.pallas import tpu_sc as plsc

_N = 10000
_E = 320000
_D = 128
_DP = 144
_NC = 2
_NS = 16
_NW = _NC * _NS
_C = 128
_NCHUNK = _E // _C
_RPT = _N // _NS


def _sc_aggregate(xa, src, dst, zinit):
    mesh = plsc.VectorSubcoreMesh(core_axis_name="c", subcore_axis_name="s")

    @functools.partial(
        pl.kernel,
        mesh=mesh,
        out_type=jax.ShapeDtypeStruct((_NC, _N, _DP), jnp.float32),
        scratch_types=[
            pltpu.VMEM((_C,), jnp.int32),
            pltpu.VMEM((_C,), jnp.int32),
            pltpu.VMEM((_C, _DP), jnp.float32),
            pltpu.VMEM_SHARED((_N, _DP), jnp.float32),
        ],
    )
    def agg(xa_hbm, src_hbm, dst_hbm, z_hbm, out_hbm, src_v, dst_v, rows_v, acc_sh):
        cid = lax.axis_index("c")
        sid = lax.axis_index("s")
        wid = sid * _NC + cid
        row0 = sid * _RPT

        pltpu.sync_copy(z_hbm, acc_sh.at[pl.ds(row0, _RPT)])
        plsc.subcore_barrier()

        @pl.loop(wid, _NCHUNK, step=_NW)
        def _(g):
            base = g * _C
            pltpu.sync_copy(src_hbm.at[pl.ds(base, _C)], src_v)
            pltpu.sync_copy(dst_hbm.at[pl.ds(base, _C)], dst_v)
            pltpu.sync_copy(xa_hbm.at[src_v], rows_v)
            pltpu.sync_copy(rows_v, acc_sh.at[dst_v], add=True)

        plsc.subcore_barrier()
        pltpu.sync_copy(acc_sh.at[pl.ds(row0, _RPT)],
                        out_hbm.at[cid, pl.ds(row0, _RPT)])

    return agg(xa, src, dst, zinit)


def _tc_finish(parts, x, W_l, b_l, W_r, W_fc, b_fc):
    def body(pp, xr, wl, bl, wr, wfc, bfc, out):
        p = pp[0] + pp[1]
        cnt = jnp.maximum(p[:, _D:_D + 1], 1.0)
        mean = p[:, :_D] / cnt
        h = (jnp.dot(mean, wl[...], preferred_element_type=jnp.float32)
             + jnp.dot(xr[...], wr[...], preferred_element_type=jnp.float32)
             + bl[...])
        h = jnp.maximum(h, 0.0)
        out[...] = jnp.dot(h, wfc[...], preferred_element_type=jnp.float32) + bfc[...]

    return pl.pallas_call(
        body,
        out_shape=jax.ShapeDtypeStruct((_N, 1), jnp.float32),
    )(parts, x, W_l, b_l, W_r, W_fc, b_fc)


def kernel(x, edge_index, W_l, b_l, W_r, W_fc, b_fc):
    src = edge_index[0]
    dst = edge_index[1]
    pad = jnp.zeros((_N, _DP - _D), jnp.float32).at[:, 0].set(1.0)
    xa = jnp.concatenate([x, pad], axis=1)
    zinit = jnp.zeros((_RPT, _DP), jnp.float32)
    parts = _sc_aggregate(xa, src, dst, zinit)
    return _tc_finish(parts, x, W_l, b_l[None, :], W_r, W_fc, b_fc[None, :])


# speedup vs baseline: 7.5108x; 7.5108x over previous
import dataclasses
import functools

import jax
import jax.numpy as jnp
from jax import lax
from jax.experimental import pallas as pl
from jax.experimental.pallas import tpu as pltpu
from jax.experimental.pallas import tpu_sc as plsc

_N = 10000
_E = 320000
_D = 128
_NC = 2
_NS = 16
_NW = _NC * _NS
_C = 128
_NCHUNK = _E // _C
_NP = 10240
_HR = _NP // _D
_RPT = _NP // _NS


def _sc_compiler_params():
    cp = pltpu.CompilerParams()
    if "needs_layout_passes" in pltpu.CompilerParams.__dataclass_fields__:
        cp = dataclasses.replace(cp, needs_layout_passes=False)
    return cp


def _sc_aggregate(x, src, dst, zfeat, zcnt, iota_hr):
    mesh = plsc.VectorSubcoreMesh(core_axis_name="c", subcore_axis_name="s")

    @functools.partial(
        pl.kernel,
        mesh=mesh,
        compiler_params=_sc_compiler_params(),
        out_type=(
            jax.ShapeDtypeStruct((_NC, _NP, _D), jnp.float32),
            jax.ShapeDtypeStruct((_NC, _HR, _D), jnp.float32),
        ),
        scratch_types=[
            pltpu.VMEM((_C,), jnp.int32),
            pltpu.VMEM((_C,), jnp.int32),
            pltpu.VMEM((_HR,), jnp.int32),
            pltpu.VMEM((_C, _D), jnp.float32),
            pltpu.VMEM((_HR, _D), jnp.float32),
            pltpu.VMEM_SHARED((_NP, _D), jnp.float32),
            pltpu.VMEM_SHARED((_HR, _D), jnp.float32),
        ],
    )
    def agg(x_hbm, src_hbm, dst_hbm, zf_hbm, zc_hbm, io_hbm,
            sum_hbm, cnt_hbm,
            src_v, dst_v, iota_v, rows_v, hist_v, acc_sh, cnt_sh):
        cid = lax.axis_index("c")
        sid = lax.axis_index("s")
        wid = sid * _NC + cid
        row0 = sid * _RPT

        pltpu.sync_copy(zf_hbm, acc_sh.at[pl.ds(row0, _RPT)])

        @pl.when(sid == 0)
        def _():
            pltpu.sync_copy(zc_hbm, cnt_sh)

        pltpu.sync_copy(io_hbm, iota_v)
        zeros16 = jnp.zeros((16,), jnp.float32)

        @pl.loop(0, _HR)
        def _(r):
            @pl.loop(0, _D, step=16)
            def _(c):
                hist_v[r, pl.ds(c, 16)] = zeros16

        plsc.subcore_barrier()

        ones16 = jnp.ones((16,), jnp.float32)

        @pl.loop(wid, _NCHUNK, step=_NW)
        def _(g):
            base = g * _C
            pltpu.sync_copy(src_hbm.at[pl.ds(base, _C)], src_v)
            pltpu.sync_copy(dst_hbm.at[pl.ds(base, _C)], dst_v)
            pltpu.sync_copy(x_hbm.at[src_v], rows_v)
            pltpu.sync_copy(rows_v, acc_sh.at[dst_v], add=True)

            @pl.loop(0, _C, step=16)
            def _(k):
                idx = dst_v[pl.ds(k, 16)]
                plsc.addupdate_scatter(hist_v, [idx >> 7, idx & 127], ones16)

        pltpu.sync_copy(hist_v, cnt_sh.at[iota_v], add=True)
        plsc.subcore_barrier()

        pltpu.sync_copy(acc_sh.at[pl.ds(row0, _RPT)],
                        sum_hbm.at[cid, pl.ds(row0, _RPT)])

        @pl.when(sid == 0)
        def _():
            pltpu.sync_copy(cnt_sh, cnt_hbm.at[cid])

    return agg(x, src, dst, zfeat, zcnt, iota_hr)


def _tc_finish(parts, cnt, x, W_l, b_l, W_r, W_fc, b_fc):
    def body(pp, cc, xr, wl, bl, wr, wfc, bfc, out):
        p = pp[0, :_N, :] + pp[1, :_N, :]
        mean = p / jnp.maximum(cc[...], 1.0)
        h = (jnp.dot(mean, wl[...], preferred_element_type=jnp.float32)
             + jnp.dot(xr[...], wr[...], preferred_element_type=jnp.float32)
             + bl[...])
        h = jnp.maximum(h, 0.0)
        out[...] = jnp.dot(h, wfc[...], preferred_element_type=jnp.float32) + bfc[...]

    return pl.pallas_call(
        body,
        out_shape=jax.ShapeDtypeStruct((_N, 1), jnp.float32),
    )(parts, cnt, x, W_l, b_l, W_r, W_fc, b_fc)


def kernel(x, edge_index, W_l, b_l, W_r, W_fc, b_fc):
    src = edge_index[0]
    dst = edge_index[1]
    zfeat = jnp.zeros((_RPT, _D), jnp.float32)
    zcnt = jnp.zeros((_HR, _D), jnp.float32)
    iota_hr = jnp.arange(_HR, dtype=jnp.int32)
    parts, cnts = _sc_aggregate(x, src, dst, zfeat, zcnt, iota_hr)
    cnt = (cnts[0] + cnts[1]).reshape(_NP, 1)[:_N]
    return _tc_finish(parts, cnt, x, W_l, b_l[None, :], W_r, W_fc, b_fc[None, :])
